# baseline (speedup 1.0000x reference)
import jax
import jax.numpy as jnp
from jax import lax
from jax.experimental import pallas as pl
from jax.experimental.pallas import tpu as pltpu
from jax.experimental.pallas import tpu_sc as plsc

_NC, _NS, _L = 2, 16, 16
_NW = _NC * _NS
_B, _N = 16, 2048
_TOK_W = (_B * _N) // _NW
_CHUNK = 64
_NCHUNK = _TOK_W // _CHUNK
_D = 256


_NBUF = 3


def _sc_body(shift_hbm, dur_hbm, pos_hbm, pe_hbm, out_hbm,
             shift_v, dur_v, pos_v, idx_v, rows0, rows1, rows2,
             gsa0, gsb0, gsa1, gsb1, gsa2, gsb2, wsem0, wsem1, wsem2):
    wid = lax.axis_index("s") * _NC + lax.axis_index("c")
    b = wid // 2
    half = wid % 2

    c_sh = pltpu.async_copy(shift_hbm, shift_v, gsa0)
    c_du = pltpu.async_copy(dur_hbm, dur_v, gsb0)
    c_po = pltpu.async_copy(pos_hbm.at[b, pl.ds(half * _TOK_W, _TOK_W)],
                            pos_v, gsa1)
    c_sh.wait()
    c_du.wait()
    c_po.wait()

    bvec = jnp.full((_L,), b, jnp.int32)
    shift_b = plsc.load_gather(shift_v, [bvec])
    dur_b = plsc.load_gather(dur_v, [bvec])
    em1 = dur_b - 1

    rows = (rows0, rows1, rows2)
    gsa = (gsa0, gsa1, gsa2)
    gsb = (gsb0, gsb1, gsb2)
    wsem = (wsem0, wsem1, wsem2)
    n0 = half * _TOK_W

    def fire_gathers(j, k):
        ga = pltpu.async_copy(pe_hbm.at[idx_v.at[j, pl.ds(0, _CHUNK)]],
                              rows[k].at[:, pl.ds(0, _D)], gsa[k])
        gb = pltpu.async_copy(pe_hbm.at[idx_v.at[j, pl.ds(_CHUNK, _CHUNK)]],
                              rows[k].at[:, pl.ds(_D, _D)], gsb[k])
        return ga, gb

    g = [None] * _NBUF
    per_chunk = _CHUNK // _L
    for i in range(_TOK_W // _L):
        p = pos_v[pl.ds(i * _L, _L)]
        s = p + shift_b
        e = em1 - s
        j = i // per_chunk
        c0 = (i % per_chunk) * _L
        idx_v[j, pl.ds(c0, _L)] = s
        idx_v[j, pl.ds(_CHUNK + c0, _L)] = e
        if i % per_chunk == per_chunk - 1 and j < _NBUF - 1:
            g[j] = fire_gathers(j, j)

    w = [None] * _NBUF
    for j in range(_NCHUNK):
        k = j % _NBUF
        nk = (j + _NBUF - 1) % _NBUF
        g[k][0].wait()
        g[k][1].wait()
        if j + _NBUF - 1 < _NCHUNK:
            if w[nk] is not None:
                w[nk].wait()
            g[nk] = fire_gathers(j + _NBUF - 1, nk)
        w[k] = pltpu.async_copy(
            rows[k], out_hbm.at[b, pl.ds(n0 + j * _CHUNK, _CHUNK)], wsem[k])
    for k in range(_NBUF):
        if w[k] is not None:
            w[k].wait()


@jax.jit
def kernel(shift_from_start, duration, pos, pe):
    mesh = plsc.VectorSubcoreMesh(
        core_axis_name="c", subcore_axis_name="s",
        num_cores=_NC, num_subcores=_NS)
    f = pl.kernel(
        _sc_body,
        out_type=jax.ShapeDtypeStruct((_B, _N, 2 * _D), jnp.float32),
        mesh=mesh,
        compiler_params=pltpu.CompilerParams(
            needs_layout_passes=False,
            disable_bounds_checks=True,
            disable_semaphore_checks=True),
        scratch_types=[
            pltpu.VMEM((_B,), jnp.int32),
            pltpu.VMEM((_B,), jnp.int32),
            pltpu.VMEM((_TOK_W,), jnp.int32),
            pltpu.VMEM((_NCHUNK, 2 * _CHUNK), jnp.int32),
            pltpu.VMEM((_CHUNK, 2 * _D), jnp.float32),
            pltpu.VMEM((_CHUNK, 2 * _D), jnp.float32),
            pltpu.VMEM((_CHUNK, 2 * _D), jnp.float32),
            pltpu.SemaphoreType.DMA,
            pltpu.SemaphoreType.DMA,
            pltpu.SemaphoreType.DMA,
            pltpu.SemaphoreType.DMA,
            pltpu.SemaphoreType.DMA,
            pltpu.SemaphoreType.DMA,
            pltpu.SemaphoreType.DMA,
            pltpu.SemaphoreType.DMA,
            pltpu.SemaphoreType.DMA,
        ],
    )
    return f(shift_from_start.astype(jnp.int32), duration.astype(jnp.int32),
             pos.astype(jnp.int32), pe)


# speedup vs baseline: 1.0265x; 1.0265x over previous
import jax
import jax.numpy as jnp
from jax import lax
from jax.experimental import pallas as pl
from jax.experimental.pallas import tpu as pltpu
from jax.experimental.pallas import tpu_sc as plsc

_NC, _NS, _L = 2, 16, 16
_NW = _NC * _NS
_B, _N = 16, 2048
_TOK_W = (_B * _N) // _NW
_CHUNK = 64
_NCHUNK = _TOK_W // _CHUNK
_D = 256
_PER_CHUNK = _CHUNK // _L


def _sc_body(shift_hbm, dur_hbm, pos_hbm, pe_hbm, out_hbm,
             shift_v, dur_v, pos_v, idx_v, rows0, rows1,
             gsa0, gsb0, gsa1, gsb1, wsem0, wsem1):
    wid = lax.axis_index("s") * _NC + lax.axis_index("c")
    b = wid // 2
    half = wid % 2

    c_sh = pltpu.async_copy(shift_hbm, shift_v, gsa0)
    c_du = pltpu.async_copy(dur_hbm, dur_v, gsb0)
    c_po = pltpu.async_copy(pos_hbm.at[b, pl.ds(half * _TOK_W, _TOK_W)],
                            pos_v, gsa1)
    c_sh.wait()
    c_du.wait()
    c_po.wait()

    bvec = jnp.full((_L,), b, jnp.int32)
    shift_b = plsc.load_gather(shift_v, [bvec])
    dur_b = plsc.load_gather(dur_v, [bvec])
    em1 = dur_b - 1

    rows = (rows0, rows1)
    gsa = (gsa0, gsa1)
    gsb = (gsb0, gsb1)
    wsem = (wsem0, wsem1)
    n0 = half * _TOK_W

    def build(i, _):
        p = pos_v[pl.ds(i * _L, _L)]
        s = p + shift_b
        e = em1 - s
        j = i // _PER_CHUNK
        c0 = (i % _PER_CHUNK) * _L
        idx_v[j, pl.ds(c0, _L)] = s
        idx_v[j, pl.ds(_CHUNK + c0, _L)] = e
        return _
    lax.fori_loop(0, _TOK_W // _L, build, None, unroll=4)

    def fire_gathers(j, k):
        ga = pltpu.async_copy(pe_hbm.at[idx_v.at[j, pl.ds(0, _CHUNK)]],
                              rows[k].at[:, pl.ds(0, _D)], gsa[k])
        gb = pltpu.async_copy(pe_hbm.at[idx_v.at[j, pl.ds(_CHUNK, _CHUNK)]],
                              rows[k].at[:, pl.ds(_D, _D)], gsb[k])
        return ga, gb

    def wait_gathers(j, k):
        pltpu.make_async_copy(pe_hbm.at[idx_v.at[j, pl.ds(0, _CHUNK)]],
                              rows[k].at[:, pl.ds(0, _D)], gsa[k]).wait()
        pltpu.make_async_copy(pe_hbm.at[idx_v.at[j, pl.ds(_CHUNK, _CHUNK)]],
                              rows[k].at[:, pl.ds(_D, _D)], gsb[k]).wait()

    def fire_write(j, k):
        return pltpu.async_copy(
            rows[k], out_hbm.at[b, pl.ds(n0 + j * _CHUNK, _CHUNK)], wsem[k])

    def wait_write(j, k):
        pltpu.make_async_copy(
            rows[k], out_hbm.at[b, pl.ds(n0 + j * _CHUNK, _CHUNK)],
            wsem[k]).wait()

    fire_gathers(0, 0)

    def ring(jj, _):
        for t in range(2):
            c = 2 * jj + t
            k = t
            nk = 1 - t
            wait_gathers(c, k)

            @pl.when(c >= 1)
            def _():
                wait_write(c - 1, nk)

            @pl.when(c + 1 < _NCHUNK)
            def _():
                fire_gathers(c + 1, nk)

            fire_write(c, k)
        return _
    lax.fori_loop(0, _NCHUNK // 2, ring, None)

    wait_write(_NCHUNK - 1, 1)


@jax.jit
def kernel(shift_from_start, duration, pos, pe):
    mesh = plsc.VectorSubcoreMesh(
        core_axis_name="c", subcore_axis_name="s",
        num_cores=_NC, num_subcores=_NS)
    f = pl.kernel(
        _sc_body,
        out_type=jax.ShapeDtypeStruct((_B, _N, 2 * _D), jnp.float32),
        mesh=mesh,
        compiler_params=pltpu.CompilerParams(
            needs_layout_passes=False,
            disable_bounds_checks=True,
            disable_semaphore_checks=True),
        scratch_types=[
            pltpu.VMEM((_B,), jnp.int32),
            pltpu.VMEM((_B,), jnp.int32),
            pltpu.VMEM((_TOK_W,), jnp.int32),
            pltpu.VMEM((_NCHUNK, 2 * _CHUNK), jnp.int32),
            pltpu.VMEM((_CHUNK, 2 * _D), jnp.float32),
            pltpu.VMEM((_CHUNK, 2 * _D), jnp.float32),
            pltpu.SemaphoreType.DMA,
            pltpu.SemaphoreType.DMA,
            pltpu.SemaphoreType.DMA,
            pltpu.SemaphoreType.DMA,
            pltpu.SemaphoreType.DMA,
            pltpu.SemaphoreType.DMA,
        ],
    )
    return f(shift_from_start.astype(jnp.int32), duration.astype(jnp.int32),
             pos.astype(jnp.int32), pe)


# speedup vs baseline: 1.0308x; 1.0042x over previous
import jax
import jax.numpy as jnp
from jax import lax
from jax.experimental import pallas as pl
from jax.experimental.pallas import tpu as pltpu
from jax.experimental.pallas import tpu_sc as plsc

_NC, _NS, _L = 2, 16, 16
_NW = _NC * _NS
_B, _N = 16, 2048
_TOK_W = (_B * _N) // _NW
_CHUNK = 64
_NCHUNK = _TOK_W // _CHUNK
_D = 256
_PER_CHUNK = _CHUNK // _L


def _sc_body(shift_hbm, dur_hbm, pos_hbm, pe_hbm, out_hbm,
             shift_v, dur_v, pos_v, idx_v, rows0, rows1,
             gsa0, gsb0, gsa1, gsb1, wsem0, wsem1):
    wid = lax.axis_index("s") * _NC + lax.axis_index("c")
    b = wid // 2
    half = wid % 2

    c_sh = pltpu.async_copy(shift_hbm, shift_v, gsa0)
    c_du = pltpu.async_copy(dur_hbm, dur_v, gsb0)
    c_po = pltpu.async_copy(pos_hbm.at[b, pl.ds(half * _TOK_W, _TOK_W)],
                            pos_v, gsa1)
    c_sh.wait()
    c_du.wait()
    c_po.wait()

    bvec = jnp.full((_L,), b, jnp.int32)
    shift_b = plsc.load_gather(shift_v, [bvec])
    dur_b = plsc.load_gather(dur_v, [bvec])
    em1 = dur_b - 1

    rows = (rows0, rows1)
    gsa = (gsa0, gsa1)
    gsb = (gsb0, gsb1)
    wsem = (wsem0, wsem1)
    n0 = half * _TOK_W

    def build(i, _):
        p = pos_v[pl.ds(i * _L, _L)]
        s = p + shift_b
        e = em1 - s
        j = i // _PER_CHUNK
        c0 = (i % _PER_CHUNK) * _L
        idx_v[j, pl.ds(c0, _L)] = s
        idx_v[j, pl.ds(_CHUNK + c0, _L)] = e
        return _

    def fire_gathers(j, k):
        ga = pltpu.async_copy(pe_hbm.at[idx_v.at[j, pl.ds(0, _CHUNK)]],
                              rows[k].at[:, pl.ds(0, _D)], gsa[k])
        gb = pltpu.async_copy(pe_hbm.at[idx_v.at[j, pl.ds(_CHUNK, _CHUNK)]],
                              rows[k].at[:, pl.ds(_D, _D)], gsb[k])
        return ga, gb

    def wait_gathers(j, k):
        pltpu.make_async_copy(pe_hbm.at[idx_v.at[j, pl.ds(0, _CHUNK)]],
                              rows[k].at[:, pl.ds(0, _D)], gsa[k]).wait()
        pltpu.make_async_copy(pe_hbm.at[idx_v.at[j, pl.ds(_CHUNK, _CHUNK)]],
                              rows[k].at[:, pl.ds(_D, _D)], gsb[k]).wait()

    def fire_write(j, k):
        return pltpu.async_copy(
            rows[k], out_hbm.at[b, pl.ds(n0 + j * _CHUNK, _CHUNK)], wsem[k])

    def wait_write(j, k):
        pltpu.make_async_copy(
            rows[k], out_hbm.at[b, pl.ds(n0 + j * _CHUNK, _CHUNK)],
            wsem[k]).wait()

    for i in range(_PER_CHUNK):
        build(i, None)
    fire_gathers(0, 0)
    lax.fori_loop(_PER_CHUNK, _TOK_W // _L, build, None, unroll=4)


    def ring(jj, _):
        for t in range(2):
            c = 2 * jj + t
            k = t
            nk = 1 - t
            wait_gathers(c, k)

            @pl.when(c >= 1)
            def _():
                wait_write(c - 1, nk)

            @pl.when(c + 1 < _NCHUNK)
            def _():
                fire_gathers(c + 1, nk)

            fire_write(c, k)
        return _
    lax.fori_loop(0, _NCHUNK // 2, ring, None)

    wait_write(_NCHUNK - 1, 1)


@jax.jit
def kernel(shift_from_start, duration, pos, pe):
    mesh = plsc.VectorSubcoreMesh(
        core_axis_name="c", subcore_axis_name="s",
        num_cores=_NC, num_subcores=_NS)
    f = pl.kernel(
        _sc_body,
        out_type=jax.ShapeDtypeStruct((_B, _N, 2 * _D), jnp.float32),
        mesh=mesh,
        compiler_params=pltpu.CompilerParams(
            needs_layout_passes=False,
            disable_bounds_checks=True,
            disable_semaphore_checks=True),
        scratch_types=[
            pltpu.VMEM((_B,), jnp.int32),
            pltpu.VMEM((_B,), jnp.int32),
            pltpu.VMEM((_TOK_W,), jnp.int32),
            pltpu.VMEM((_NCHUNK, 2 * _CHUNK), jnp.int32),
            pltpu.VMEM((_CHUNK, 2 * _D), jnp.float32),
            pltpu.VMEM((_CHUNK, 2 * _D), jnp.float32),
            pltpu.SemaphoreType.DMA,
            pltpu.SemaphoreType.DMA,
            pltpu.SemaphoreType.DMA,
            pltpu.SemaphoreType.DMA,
            pltpu.SemaphoreType.DMA,
            pltpu.SemaphoreType.DMA,
        ],
    )
    return f(shift_from_start.astype(jnp.int32), duration.astype(jnp.int32),
             pos.astype(jnp.int32), pe)
